# baseline (speedup 1.0000x reference)
import functools

import jax
import jax.numpy as jnp
from jax import lax
from jax.experimental import pallas as pl
from jax.experimental.pallas import tpu as pltpu
from jax.experimental.pallas import tpu_sc as plsc

K = 9
SUB = 16

_HI = jax.lax.Precision.HIGHEST

_NC = 2
_NS = 16
_LANES = 16

_CH = 64
_GROWS = 96


def _fold_kernel(fc2_ref, cw_ref, wf_ref):
    fc2 = fc2_ref[...]
    for k in range(K):
        wf_ref[k] = jnp.dot(fc2, cw_ref[k], precision=_HI,
                            preferred_element_type=jnp.float32)


def _tc_kernel(x_ref, fc1_ref, ss_ref, sb_ref, wf_ref, idx_ref, vt_ref):
    b = pl.program_id(0)
    x = x_ref[0]
    fc1 = fc1_ref[...]
    s = ss_ref[0, 0]
    bias = sb_ref[0, 0]
    C2 = fc1.shape[0]
    N = x.shape[1]
    M = N // SUB

    h = jnp.dot(fc1.astype(jnp.bfloat16), x.astype(jnp.bfloat16),
                preferred_element_type=jnp.float32)
    h = s * jnp.square(jnp.maximum(h, 0.0)) + bias

    row = jax.lax.broadcasted_iota(jnp.int32, (N, M), 0)
    col = jax.lax.broadcasted_iota(jnp.int32, (N, M), 1)
    sel_nm = (row == col * SUB).astype(jnp.float32)
    cand = jnp.dot(h, sel_nm, precision=_HI,
                   preferred_element_type=jnp.float32)

    n2 = jnp.sum(h * h, axis=0, keepdims=True)
    csq = cand * cand
    ones = jnp.ones((C2, 1), jnp.float32)
    c2 = jax.lax.dot_general(csq, ones, (((0,), (0,)), ((), ())),
                             precision=_HI,
                             preferred_element_type=jnp.float32)
    d = jax.lax.dot_general(cand.astype(jnp.bfloat16), h.astype(jnp.bfloat16),
                            (((0,), (0,)), ((), ())),
                            preferred_element_type=jnp.float32)
    neg = 2.0 * d - c2 - n2

    iota = jax.lax.broadcasted_iota(jnp.int32, (M, N), 0)
    for k in range(K):
        mx = jnp.max(neg, axis=0, keepdims=True)
        ismax = neg >= mx
        sel = jnp.min(jnp.where(ismax, iota, M), axis=0, keepdims=True)
        idx_ref[0, pl.ds(k, 1), :] = sel + (b * (K * M) + k * M)
        onehot = iota == sel
        neg = jnp.where(onehot, -jnp.inf, neg)
        vt_k = jax.lax.dot_general(cand, wf_ref[k], (((0,), (1,)), ((), ())),
                                   precision=_HI,
                                   preferred_element_type=jnp.float32)
        pad = vt_ref.shape[2] - vt_k.shape[1]
        vt_k = jnp.concatenate([vt_k, jnp.zeros((M, pad), jnp.float32)], axis=1)
        vt_ref[0, pl.ds(k * M, M), :] = vt_k


def _make_sc_gather(BN, C, CP):
    NW = _NC * _NS
    per_w = BN // NW
    n_chunks = per_w // _CH
    G = (_CH * K) // _GROWS
    mesh = plsc.VectorSubcoreMesh(core_axis_name="c", subcore_axis_name="s")

    @functools.partial(
        pl.kernel,
        out_type=jax.ShapeDtypeStruct((BN, C), jnp.float32),
        mesh=mesh,
        scratch_types=[
            pltpu.VMEM((G, _GROWS), jnp.int32),
            pltpu.VMEM((_CH * K, CP), jnp.float32),
            pltpu.VMEM((_CH, C), jnp.float32),
            pltpu.SemaphoreType.DMA,
        ],
    )
    def sc_gather(idx_hbm, table_hbm, out_hbm, idx_v, rows_v, acc_v, sem):
        wid = lax.axis_index("s") * _NC + lax.axis_index("c")
        base = wid * per_w

        def chunk_body(ci, carry):
            start = base + ci * _CH
            pltpu.sync_copy(idx_hbm.at[wid * n_chunks + ci], idx_v)
            copies = [
                pltpu.async_copy(table_hbm.at[idx_v.at[g]],
                                 rows_v.at[pl.ds(g * _GROWS, _GROWS)], sem)
                for g in range(G)
            ]
            for c in copies:
                c.wait()

            def node_body(i, carry2):
                for c in range(C // _LANES):
                    sl = pl.ds(c * _LANES, _LANES)
                    acc = rows_v[i * K, sl]
                    for k in range(1, K):
                        acc = acc + rows_v[i * K + k, sl]
                    acc_v[i, sl] = acc
                return carry2

            lax.fori_loop(0, _CH, node_body, 0)
            pltpu.sync_copy(acc_v, out_hbm.at[pl.ds(start, _CH)])
            return carry

        lax.fori_loop(0, n_chunks, chunk_body, 0)

    return sc_gather


def kernel(x, fc1_w, star_scale, star_bias, conv_w, fc2_w):
    B, C, H, W = x.shape
    N = H * W
    BN = B * N
    C2 = fc1_w.shape[0]
    M = N // SUB
    CP = 128
    xf = x.reshape(B, C, N)
    cw = jnp.transpose(conv_w, (2, 0, 1))

    wf = pl.pallas_call(
        _fold_kernel,
        out_shape=jax.ShapeDtypeStruct((K, C, C2), jnp.float32),
    )(fc2_w, cw)

    ss = jnp.reshape(star_scale, (1, 1)).astype(jnp.float32)
    sb = jnp.reshape(star_bias, (1, 1)).astype(jnp.float32)

    idx, vt = pl.pallas_call(
        _tc_kernel,
        grid=(B,),
        in_specs=[
            pl.BlockSpec((1, C, N), lambda i: (i, 0, 0)),
            pl.BlockSpec((C2, C), lambda i: (0, 0)),
            pl.BlockSpec((1, 1), lambda i: (0, 0)),
            pl.BlockSpec((1, 1), lambda i: (0, 0)),
            pl.BlockSpec((K, C, C2), lambda i: (0, 0, 0)),
        ],
        out_specs=[
            pl.BlockSpec((1, K, N), lambda i: (i, 0, 0)),
            pl.BlockSpec((1, K * M, CP), lambda i: (i, 0, 0)),
        ],
        out_shape=[
            jax.ShapeDtypeStruct((B, K, N), jnp.int32),
            jax.ShapeDtypeStruct((B, K * M, CP), jnp.float32),
        ],
    )(xf, fc1_w, ss, sb, wf)

    n_chunks_total = BN // _CH
    idx_flat = jnp.transpose(idx, (0, 2, 1)).reshape(
        n_chunks_total, (_CH * K) // _GROWS, _GROWS)
    table = vt.reshape(B * K * M, CP)

    out_nodes = _make_sc_gather(BN, C, CP)(idx_flat, table)
    out = jnp.transpose(out_nodes.reshape(B, N, C), (0, 2, 1))
    return out.reshape(B, C, H, W)
